# baseline (speedup 1.0000x reference)
import jax
import jax.numpy as jnp
import numpy as np
from jax.experimental import pallas as pl
from jax.experimental.pallas import tpu as pltpu

_B = 8192
_D = 128
_T = 8
_K = 2
_L = 3
_NF = 8
_VR = 16
_NP = _VR * _VR

_DNT = (((1,), (1,)), ((), ()))


def _gelu(x):
    return x * 0.5 * (1.0 + jax.lax.erf(x * np.float32(1.0 / np.sqrt(2.0))))


def _ln(x, g, b):
    m = jnp.mean(x, axis=-1, keepdims=True)
    xc = x - m
    v = jnp.mean(xc * xc, axis=-1, keepdims=True)
    return xc * jax.lax.rsqrt(v + 1e-5) * g + b


def _net_kernel(a_ref, b_ref, Wi_ref, bi_ref, lig_ref, lib_ref,
                WrT_ref, br_ref, W1_ref, b1_ref, W2_ref, b2_ref,
                lng_ref, lnb_ref, Ws1T_ref, bs1_ref, Ws2T_ref, bs2_ref,
                Wd1T_ref, bd1_ref, Wd2T_ref, bd2_ref,
                sumT_ref, diffT_ref, routT_ref, aux_ref,
                tabT_ref):
    pair = jax.lax.broadcasted_iota(jnp.int32, (_NP, 1), 0)
    pa = (pair // _VR).astype(jnp.float32)
    pb = (pair % _VR).astype(jnp.float32)
    ci = jax.lax.broadcasted_iota(jnp.int32, (_NP, 4 * _NF), 1)
    freqs = jnp.exp2((ci & (_NF - 1)).astype(jnp.float32)) * np.float32(
        2.0 * np.pi / _VR)
    val = jnp.where(ci < 2 * _NF, pa, pb)
    shift = jnp.where((ci & _NF) == _NF, np.float32(np.pi / 2.0),
                      np.float32(0.0))
    x0 = jnp.sin(val * freqs + shift)

    bf = jnp.bfloat16
    x = jnp.dot(x0, Wi_ref[...],
                preferred_element_type=jnp.float32)
    x = _gelu(_ln(x + bi_ref[...], lig_ref[...], lib_ref[...]))

    ti = jax.lax.broadcasted_iota(jnp.int32, (_NP, _T), 1)
    expand = (jax.lax.broadcasted_iota(jnp.int32, (_T, _T * _D), 1) // _D
              == jax.lax.broadcasted_iota(jnp.int32, (_T, _T * _D), 0)
              ).astype(jnp.float32)
    i1 = i2 = None
    ps_list = []
    oh_list = []
    for l in range(_L):
        x16 = x
        logits = jax.lax.dot_general(
            x16, WrT_ref[l], _DNT,
            preferred_element_type=jnp.float32)
        logits = logits + br_ref[l]
        p = logits - jnp.max(logits, axis=1, keepdims=True)
        p = jnp.exp(p)
        p = p / jnp.sum(p, axis=1, keepdims=True)

        m1 = jnp.max(p, axis=1, keepdims=True)
        i1 = jnp.min(jnp.where(p == m1, ti, _T), axis=1, keepdims=True)
        p_rest = jnp.where(ti == i1, -jnp.inf, p)
        m2 = jnp.max(p_rest, axis=1, keepdims=True)
        i2 = jnp.min(jnp.where(p_rest == m2, ti, _T), axis=1, keepdims=True)
        denom = 1.0 / (m1 + m2 + 1e-9)
        oh = ((ti == i1) | (ti == i2)).astype(jnp.float32)
        gates_full = jnp.where(ti == i1, m1 * denom, 0.0) + jnp.where(
            ti == i2, m2 * denom, 0.0)
        ps_list.append(p)
        oh_list.append(oh)

        h_all = jnp.concatenate(
            [jnp.dot(x16, W1_ref[l, t],
                     preferred_element_type=jnp.float32)
             + b1_ref[l, t] for t in range(_T)], axis=1)
        h_all = _gelu(h_all)
        gw = jnp.dot(gates_full, expand, preferred_element_type=jnp.float32,
                     precision=jax.lax.Precision.HIGHEST)
        out = jnp.zeros((_NP, _D), jnp.float32)
        for t in range(_T):
            eo = jnp.dot(h_all[:, t * _D:(t + 1) * _D],
                         W2_ref[l, t],
                         preferred_element_type=jnp.float32)
            eo = eo + b2_ref[l, t]
            out = out + gw[:, t * _D:(t + 1) * _D] * eo
        x = _ln(x + out, lng_ref[l], lnb_ref[l])

    x16 = x
    hs = _gelu(jax.lax.dot_general(
        x16, Ws1T_ref[...], _DNT,
        preferred_element_type=jnp.float32) + bs1_ref[...])
    sl = jax.lax.dot_general(
        hs, Ws2T_ref[...], _DNT,
        preferred_element_type=jnp.float32)
    hd = _gelu(jax.lax.dot_general(
        x16, Wd1T_ref[...], _DNT,
        preferred_element_type=jnp.float32) + bd1_ref[...])
    dl = jax.lax.dot_general(
        hd, Wd2T_ref[...], _DNT,
        preferred_element_type=jnp.float32)
    tab = jnp.concatenate(
        [sl + bs2_ref[...], dl + bd2_ref[...],
         i1.astype(jnp.float32), i2.astype(jnp.float32),
         jnp.zeros((_NP, 3), jnp.float32)] + ps_list + oh_list, axis=1)
    tabT_ref[...] = tab.T

    pair_row = (a_ref[...] * _VR + b_ref[...])[None, :]
    onehotT = (pair_row == jax.lax.broadcasted_iota(
        jnp.int32, (_NP, _B), 0)).astype(jnp.float32)
    gT = jnp.dot(tabT_ref[...], onehotT, preferred_element_type=jnp.float32,
                 precision=jax.lax.Precision.HIGHEST)
    sumT_ref[...] = gT[0:5, :]
    diffT_ref[...] = gT[5:11, :]
    routT_ref[...] = gT[11:13, :].astype(jnp.int32)

    ps_sum = jnp.sum(gT[16:16 + _L * _T, :], axis=1, keepdims=True)
    ls_sum = jnp.sum(gT[16 + _L * _T:16 + 2 * _L * _T, :], axis=1,
                     keepdims=True)
    scale = np.float32(_T) / np.float32(_B * _B)
    aux_ref[...] = jnp.sum(ps_sum * ls_sum, keepdims=True).reshape(
        1, 1) * scale


@jax.jit
def _run(a, b, params):
    p = params

    full = lambda s: pl.BlockSpec(s, lambda: (0,) * len(s))
    in_specs = [
        full((_B,)), full((_B,)),
        full((4 * _NF, _D)), full((_D,)), full((_D,)), full((_D,)),
        full((_L, _T, _D)), full((_L, _T)),
        full((_L, _T, _D, _D)), full((_L, _T, _D)),
        full((_L, _T, _D, _D)), full((_L, _T, _D)),
        full((_L, _D)), full((_L, _D)),
        full((_D // 2, _D)), full((_D // 2,)),
        full((5, _D // 2)), full((5,)),
        full((_D // 2, _D)), full((_D // 2,)),
        full((6, _D // 2)), full((6,)),
    ]
    out_specs = [
        full((5, _B)), full((6, _B)), full((_K, _B)), full((1, 1)),
    ]
    out_shape = [
        jax.ShapeDtypeStruct((5, _B), jnp.float32),
        jax.ShapeDtypeStruct((6, _B), jnp.float32),
        jax.ShapeDtypeStruct((_K, _B), jnp.int32),
        jax.ShapeDtypeStruct((1, 1), jnp.float32),
    ]
    slT, dlT, routT, aux = pl.pallas_call(
        _net_kernel,
        in_specs=in_specs,
        out_specs=out_specs,
        out_shape=out_shape,
        scratch_shapes=[
            pltpu.VMEM((16 + 2 * _L * _T, _NP), jnp.float32),
        ],
    )(a.astype(jnp.int32), b.astype(jnp.int32),
      p["Wi"], p["bi"], p["ln_in_g"], p["ln_in_b"],
      jnp.swapaxes(p["Wr"], 1, 2), p["br"],
      p["W1"], p["b1"], p["W2"], p["b2"],
      p["ln_g"], p["ln_b"],
      p["Ws1"].T, p["bs1"], p["Ws2"].T, p["bs2"],
      p["Wd1"].T, p["bd1"], p["Wd2"].T, p["bd2"])
    return slT.T, dlT.T, routT.T, aux[0, 0]


def kernel(a, b, params):
    return _run(a, b, params)
